# baseline (speedup 1.0000x reference)
import functools

import jax
import jax.numpy as jnp
from jax import lax
from jax.experimental import pallas as pl
from jax.experimental.pallas import tpu as pltpu
from jax.experimental.pallas import tpu_sc as plsc

N = 10000
NP = 10240
E = 160000
NWORDS = NP * NP
PAD = 64
NWK = 32
Q = NWORDS // NWK
ZB = 65536
CH = 4000
CAP = 8192
BK = 1024
KB = NP // BK


def _sc_build_body(src_hbm, dst_hbm, a_hbm, zbuf, sbuf, dbuf, ibuf, obuf, sem):
    c = lax.axis_index("c")
    s = lax.axis_index("s")
    wid = s * 2 + c
    lo = wid * Q
    hi = lo + Q

    zero16 = jnp.zeros((16,), jnp.float32)
    one16 = jnp.ones((16,), jnp.float32)

    def fill_z(i, _):
        zbuf[pl.ds(i * 16, 16)] = zero16
        return 0

    lax.fori_loop(0, ZB // 16, fill_z, 0)

    def fill_o(i, _):
        obuf[pl.ds(i * 16, 16)] = one16
        return 0

    lax.fori_loop(0, CH // 16, fill_o, 0)

    def zc(i, _):
        pltpu.sync_copy(zbuf, a_hbm.at[pl.ds(lo + i * ZB, ZB)])
        return 0

    lax.fori_loop(0, Q // ZB, zc, 0)

    def chunk(ci, _):
        base = ci * CH
        pltpu.sync_copy(src_hbm.at[pl.ds(base, CH)], sbuf)
        pltpu.sync_copy(dst_hbm.at[pl.ds(base, CH)], dbuf)

        def step(t, _):
            sv = sbuf[pl.ds(t * 16, 16)]
            dv = dbuf[pl.ds(t * 16, 16)]
            flat = sv * NP + dv
            m = (flat >= lo) & (flat < hi) & (sv != dv)
            ibuf[pl.ds(t * 16, 16)] = jnp.where(m, flat, NWORDS)
            return 0

        lax.fori_loop(0, CH // 16, step, 0)
        pltpu.async_copy(obuf, a_hbm.at[ibuf], sem).wait()
        return 0

    lax.fori_loop(0, E // CH, chunk, 0)


@functools.cache
def _sc_build():
    return pl.kernel(
        _sc_build_body,
        out_type=jax.ShapeDtypeStruct((NWORDS + PAD,), jnp.float32),
        mesh=plsc.VectorSubcoreMesh(core_axis_name="c", subcore_axis_name="s"),
        scratch_types=[
            pltpu.VMEM((ZB,), jnp.float32),
            pltpu.VMEM((CH,), jnp.int32),
            pltpu.VMEM((CH,), jnp.int32),
            pltpu.VMEM((CH,), jnp.int32),
            pltpu.VMEM((CH,), jnp.float32),
            pltpu.SemaphoreType.DMA,
        ],
    )


def _dinv(col):
    return jnp.where(col > 0.0, 1.0 / jnp.sqrt(jnp.maximum(col, 1.0)), 0.0)


def _conv_body(a_ref, ab_ref, deg_ref):
    t = a_ref[...]
    ab_ref[...] = t.astype(jnp.bfloat16)
    rs = jnp.sum(t, axis=1)
    deg_ref[...] = jnp.broadcast_to(rs[:, None], deg_ref.shape)


def _h_body(x_ref, w1_ref, deg_ref, h_ref, hp_ref):
    h = jnp.maximum(jnp.dot(x_ref[...], w1_ref[...],
                            preferred_element_type=jnp.float32), 0.0)
    h_ref[...] = h
    d1 = _dinv(deg_ref[:, 0:1])
    hp_ref[...] = (d1 * h).astype(jnp.bfloat16)


def _mm_body(a_ik, a_kj, hp1_k, b_out, deg2_out, h1_out, acc):
    i = pl.program_id(0)
    j = pl.program_id(1)
    k = pl.program_id(2)
    kn = pl.num_programs(2)

    @pl.when(k == 0)
    def _():
        acc[...] = jnp.zeros_like(acc)

    acc[...] += jnp.dot(a_ik[...], a_kj[...],
                        preferred_element_type=jnp.float32)

    @pl.when(j == 0)
    def _():
        part = jnp.dot(a_ik[...], hp1_k[...],
                       preferred_element_type=jnp.float32)
        prev = jnp.where(k == 0, jnp.zeros_like(part), h1_out[...])
        h1_out[...] = prev + part

    @pl.when(k == kn - 1)
    def _():
        cval = acc[...]
        bt = (cval > 0.0).astype(jnp.float32)
        ri = lax.broadcasted_iota(jnp.int32, bt.shape, 0)
        ci = lax.broadcasted_iota(jnp.int32, bt.shape, 1)
        bt = jnp.where((ri == ci) & (i == j), 0.0, bt)
        b_out[...] = bt.astype(jnp.bfloat16)
        rs = jnp.sum(bt, axis=1)
        prev = jnp.where(j == 0, jnp.zeros_like(deg2_out[...]), deg2_out[...])
        deg2_out[...] = prev + jnp.broadcast_to(rs[:, None], deg2_out.shape)


def _hp2_body(h_ref, deg_ref, hp_ref):
    d2 = _dinv(deg_ref[:, 0:1])
    hp_ref[...] = (d2 * h_ref[...]).astype(jnp.bfloat16)


def _h2_body(b_ik, hp2_k, h2_out, acc):
    k = pl.program_id(1)
    kn = pl.num_programs(1)

    @pl.when(k == 0)
    def _():
        acc[...] = jnp.zeros_like(acc)

    acc[...] += jnp.dot(b_ik[...], hp2_k[...],
                        preferred_element_type=jnp.float32)

    @pl.when(k == kn - 1)
    def _():
        h2_out[...] = acc[...]


def _final_body(h_ref, h1_ref, h2_ref, d1_ref, d2_ref, wf_ref, o_ref):
    wf = wf_ref[...]
    d1 = _dinv(d1_ref[:, 0:1])
    d2 = _dinv(d2_ref[:, 0:1])
    out = jnp.dot(h_ref[...], wf[0:64], preferred_element_type=jnp.float32)
    out += jnp.dot(d1 * h1_ref[...], wf[64:128],
                   preferred_element_type=jnp.float32)
    out += jnp.dot(d2 * h2_ref[...], wf[128:192],
                   preferred_element_type=jnp.float32)
    o_ref[...] = out


def kernel(x, edge_index, W1, W_final):
    src = edge_index[0].astype(jnp.int32)
    dst = edge_index[1].astype(jnp.int32)

    a_flat = _sc_build()(src, dst)
    a2d = a_flat[:NWORDS].reshape(NP, NP)

    abf, deg1 = pl.pallas_call(
        _conv_body,
        grid=(NP // 128,),
        in_specs=[pl.BlockSpec((128, NP), lambda i: (i, 0))],
        out_specs=[
            pl.BlockSpec((128, NP), lambda i: (i, 0)),
            pl.BlockSpec((128, 128), lambda i: (i, 0)),
        ],
        out_shape=[
            jax.ShapeDtypeStruct((NP, NP), jnp.bfloat16),
            jax.ShapeDtypeStruct((NP, 128), jnp.float32),
        ],
    )(a2d)

    x_pad = jnp.zeros((NP, x.shape[1]), jnp.float32).at[:N].set(x)

    h, hp1 = pl.pallas_call(
        _h_body,
        grid=(KB,),
        in_specs=[
            pl.BlockSpec((BK, 128), lambda i: (i, 0)),
            pl.BlockSpec((128, 64), lambda i: (0, 0)),
            pl.BlockSpec((BK, 128), lambda i: (i, 0)),
        ],
        out_specs=[
            pl.BlockSpec((BK, 64), lambda i: (i, 0)),
            pl.BlockSpec((BK, 64), lambda i: (i, 0)),
        ],
        out_shape=[
            jax.ShapeDtypeStruct((NP, 64), jnp.float32),
            jax.ShapeDtypeStruct((NP, 64), jnp.bfloat16),
        ],
    )(x_pad, W1, deg1)

    b2, deg2, h1raw = pl.pallas_call(
        _mm_body,
        grid=(KB, KB, KB),
        in_specs=[
            pl.BlockSpec((BK, BK), lambda i, j, k: (i, k)),
            pl.BlockSpec((BK, BK), lambda i, j, k: (k, j)),
            pl.BlockSpec((BK, 64), lambda i, j, k: (k, 0)),
        ],
        out_specs=[
            pl.BlockSpec((BK, BK), lambda i, j, k: (i, j)),
            pl.BlockSpec((BK, 128), lambda i, j, k: (i, 0)),
            pl.BlockSpec((BK, 64), lambda i, j, k: (i, 0)),
        ],
        out_shape=[
            jax.ShapeDtypeStruct((NP, NP), jnp.bfloat16),
            jax.ShapeDtypeStruct((NP, 128), jnp.float32),
            jax.ShapeDtypeStruct((NP, 64), jnp.float32),
        ],
        scratch_shapes=[pltpu.VMEM((BK, BK), jnp.float32)],
        compiler_params=pltpu.CompilerParams(
            dimension_semantics=("parallel", "arbitrary", "arbitrary")),
    )(abf, abf, hp1)

    hp2 = pl.pallas_call(
        _hp2_body,
        grid=(KB,),
        in_specs=[
            pl.BlockSpec((BK, 64), lambda i: (i, 0)),
            pl.BlockSpec((BK, 128), lambda i: (i, 0)),
        ],
        out_specs=pl.BlockSpec((BK, 64), lambda i: (i, 0)),
        out_shape=jax.ShapeDtypeStruct((NP, 64), jnp.bfloat16),
    )(h, deg2)

    h2raw = pl.pallas_call(
        _h2_body,
        grid=(KB, KB),
        in_specs=[
            pl.BlockSpec((BK, BK), lambda i, k: (i, k)),
            pl.BlockSpec((BK, 64), lambda i, k: (k, 0)),
        ],
        out_specs=pl.BlockSpec((BK, 64), lambda i, k: (i, 0)),
        out_shape=jax.ShapeDtypeStruct((NP, 64), jnp.float32),
        scratch_shapes=[pltpu.VMEM((BK, 64), jnp.float32)],
        compiler_params=pltpu.CompilerParams(
            dimension_semantics=("parallel", "arbitrary")),
    )(b2, hp2)

    out = pl.pallas_call(
        _final_body,
        grid=(KB,),
        in_specs=[
            pl.BlockSpec((BK, 64), lambda i: (i, 0)),
            pl.BlockSpec((BK, 64), lambda i: (i, 0)),
            pl.BlockSpec((BK, 64), lambda i: (i, 0)),
            pl.BlockSpec((BK, 128), lambda i: (i, 0)),
            pl.BlockSpec((BK, 128), lambda i: (i, 0)),
            pl.BlockSpec((192, 64), lambda i: (0, 0)),
        ],
        out_specs=pl.BlockSpec((BK, 64), lambda i: (i, 0)),
        out_shape=jax.ShapeDtypeStruct((NP, 64), jnp.float32),
    )(h, h1raw, h2raw, deg1, deg2, W_final)

    return out[:N]


# speedup vs baseline: 58.1869x; 58.1869x over previous
import functools

import jax
import jax.numpy as jnp
from jax import lax
from jax.experimental import pallas as pl
from jax.experimental.pallas import tpu as pltpu
from jax.experimental.pallas import tpu_sc as plsc

N = 10000
NP = 10240
E = 160000
NWORDS = NP * NP
PAD = 64
NWK = 32
Q = NWORDS // NWK
ZB = 65536
CH = 4000
CAP = 8192
BK = 1024
KB = NP // BK


def _sc_build_body(src_hbm, dst_hbm, a_hbm, zbuf, sbuf, dbuf, ibuf, obuf, sem):
    c = lax.axis_index("c")
    s = lax.axis_index("s")
    wid = s * 2 + c
    lo = wid * Q
    hi = lo + Q

    zero16 = jnp.zeros((16,), jnp.float32)

    def fill_z(i, _):
        zbuf[pl.ds(i * 16, 16)] = zero16
        return 0

    lax.fori_loop(0, ZB // 16, fill_z, 0)

    lane = lax.iota(jnp.int32, 16)

    def zc(i, _):
        pltpu.sync_copy(zbuf, a_hbm.at[pl.ds(lo + i * ZB, ZB)])
        return 0

    lax.fori_loop(0, Q // ZB, zc, 0)

    def chunk(ci, _):
        base = ci * CH
        pltpu.sync_copy(src_hbm.at[pl.ds(base, CH)], sbuf)
        pltpu.sync_copy(dst_hbm.at[pl.ds(base, CH)], dbuf)

        def step(t, _):
            sv = sbuf[pl.ds(t * 16, 16)]
            dv = dbuf[pl.ds(t * 16, 16)]
            flat = sv * NP + dv
            m = (flat >= lo) & (flat < hi) & (sv != dv)
            dummy = lo + t * NP + N + lane * 8
            ibuf[pl.ds(t * 16, 16)] = jnp.where(m, flat, dummy)
            obuf[pl.ds(t * 16, 16)] = jnp.where(m, 1.0, 0.0)
            return 0

        lax.fori_loop(0, CH // 16, step, 0)
        pltpu.async_copy(obuf, a_hbm.at[ibuf], sem).wait()
        return 0

    lax.fori_loop(0, E // CH, chunk, 0)


@functools.cache
def _sc_build():
    return pl.kernel(
        _sc_build_body,
        out_type=jax.ShapeDtypeStruct((NWORDS + PAD,), jnp.float32),
        mesh=plsc.VectorSubcoreMesh(core_axis_name="c", subcore_axis_name="s"),
        scratch_types=[
            pltpu.VMEM((ZB,), jnp.float32),
            pltpu.VMEM((CH,), jnp.int32),
            pltpu.VMEM((CH,), jnp.int32),
            pltpu.VMEM((CH,), jnp.int32),
            pltpu.VMEM((CH,), jnp.float32),
            pltpu.SemaphoreType.DMA,
        ],
    )


def _dinv(col):
    return jnp.where(col > 0.0, 1.0 / jnp.sqrt(jnp.maximum(col, 1.0)), 0.0)


def _conv_body(a_ref, ab_ref, deg_ref):
    t = a_ref[...]
    ab_ref[...] = t.astype(jnp.bfloat16)
    rs = jnp.sum(t, axis=1)
    deg_ref[...] = jnp.broadcast_to(rs[:, None], deg_ref.shape)


def _h_body(x_ref, w1_ref, deg_ref, h_ref, hp_ref):
    h = jnp.maximum(jnp.dot(x_ref[...], w1_ref[...],
                            preferred_element_type=jnp.float32), 0.0)
    h_ref[...] = h
    d1 = _dinv(deg_ref[:, 0:1])
    hp_ref[...] = (d1 * h).astype(jnp.bfloat16)


def _mm_body(a_ik, a_kj, hp1_k, b_out, deg2_out, h1_out, acc):
    i = pl.program_id(0)
    j = pl.program_id(1)
    k = pl.program_id(2)
    kn = pl.num_programs(2)

    @pl.when(k == 0)
    def _():
        acc[...] = jnp.zeros_like(acc)

    acc[...] += jnp.dot(a_ik[...], a_kj[...],
                        preferred_element_type=jnp.float32)

    @pl.when(j == 0)
    def _():
        part = jnp.dot(a_ik[...], hp1_k[...],
                       preferred_element_type=jnp.float32)
        prev = jnp.where(k == 0, jnp.zeros_like(part), h1_out[...])
        h1_out[...] = prev + part

    @pl.when(k == kn - 1)
    def _():
        cval = acc[...]
        bt = (cval > 0.0).astype(jnp.float32)
        ri = lax.broadcasted_iota(jnp.int32, bt.shape, 0)
        ci = lax.broadcasted_iota(jnp.int32, bt.shape, 1)
        bt = jnp.where((ri == ci) & (i == j), 0.0, bt)
        b_out[...] = bt.astype(jnp.bfloat16)
        rs = jnp.sum(bt, axis=1)
        prev = jnp.where(j == 0, jnp.zeros_like(deg2_out[...]), deg2_out[...])
        deg2_out[...] = prev + jnp.broadcast_to(rs[:, None], deg2_out.shape)


def _hp2_body(h_ref, deg_ref, hp_ref):
    d2 = _dinv(deg_ref[:, 0:1])
    hp_ref[...] = (d2 * h_ref[...]).astype(jnp.bfloat16)


def _h2_body(b_ik, hp2_k, h2_out, acc):
    k = pl.program_id(1)
    kn = pl.num_programs(1)

    @pl.when(k == 0)
    def _():
        acc[...] = jnp.zeros_like(acc)

    acc[...] += jnp.dot(b_ik[...], hp2_k[...],
                        preferred_element_type=jnp.float32)

    @pl.when(k == kn - 1)
    def _():
        h2_out[...] = acc[...]


def _final_body(h_ref, h1_ref, h2_ref, d1_ref, d2_ref, wf_ref, o_ref):
    wf = wf_ref[...]
    d1 = _dinv(d1_ref[:, 0:1])
    d2 = _dinv(d2_ref[:, 0:1])
    out = jnp.dot(h_ref[...], wf[0:64], preferred_element_type=jnp.float32)
    out += jnp.dot(d1 * h1_ref[...], wf[64:128],
                   preferred_element_type=jnp.float32)
    out += jnp.dot(d2 * h2_ref[...], wf[128:192],
                   preferred_element_type=jnp.float32)
    o_ref[...] = out


def kernel(x, edge_index, W1, W_final):
    src = edge_index[0].astype(jnp.int32)
    dst = edge_index[1].astype(jnp.int32)

    a_flat = _sc_build()(src, dst)
    a2d = a_flat[:NWORDS].reshape(NP, NP)

    abf, deg1 = pl.pallas_call(
        _conv_body,
        grid=(NP // 128,),
        in_specs=[pl.BlockSpec((128, NP), lambda i: (i, 0))],
        out_specs=[
            pl.BlockSpec((128, NP), lambda i: (i, 0)),
            pl.BlockSpec((128, 128), lambda i: (i, 0)),
        ],
        out_shape=[
            jax.ShapeDtypeStruct((NP, NP), jnp.bfloat16),
            jax.ShapeDtypeStruct((NP, 128), jnp.float32),
        ],
    )(a2d)

    x_pad = jnp.zeros((NP, x.shape[1]), jnp.float32).at[:N].set(x)

    h, hp1 = pl.pallas_call(
        _h_body,
        grid=(KB,),
        in_specs=[
            pl.BlockSpec((BK, 128), lambda i: (i, 0)),
            pl.BlockSpec((128, 64), lambda i: (0, 0)),
            pl.BlockSpec((BK, 128), lambda i: (i, 0)),
        ],
        out_specs=[
            pl.BlockSpec((BK, 64), lambda i: (i, 0)),
            pl.BlockSpec((BK, 64), lambda i: (i, 0)),
        ],
        out_shape=[
            jax.ShapeDtypeStruct((NP, 64), jnp.float32),
            jax.ShapeDtypeStruct((NP, 64), jnp.bfloat16),
        ],
    )(x_pad, W1, deg1)

    b2, deg2, h1raw = pl.pallas_call(
        _mm_body,
        grid=(KB, KB, KB),
        in_specs=[
            pl.BlockSpec((BK, BK), lambda i, j, k: (i, k)),
            pl.BlockSpec((BK, BK), lambda i, j, k: (k, j)),
            pl.BlockSpec((BK, 64), lambda i, j, k: (k, 0)),
        ],
        out_specs=[
            pl.BlockSpec((BK, BK), lambda i, j, k: (i, j)),
            pl.BlockSpec((BK, 128), lambda i, j, k: (i, 0)),
            pl.BlockSpec((BK, 64), lambda i, j, k: (i, 0)),
        ],
        out_shape=[
            jax.ShapeDtypeStruct((NP, NP), jnp.bfloat16),
            jax.ShapeDtypeStruct((NP, 128), jnp.float32),
            jax.ShapeDtypeStruct((NP, 64), jnp.float32),
        ],
        scratch_shapes=[pltpu.VMEM((BK, BK), jnp.float32)],
        compiler_params=pltpu.CompilerParams(
            dimension_semantics=("parallel", "arbitrary", "arbitrary")),
    )(abf, abf, hp1)

    hp2 = pl.pallas_call(
        _hp2_body,
        grid=(KB,),
        in_specs=[
            pl.BlockSpec((BK, 64), lambda i: (i, 0)),
            pl.BlockSpec((BK, 128), lambda i: (i, 0)),
        ],
        out_specs=pl.BlockSpec((BK, 64), lambda i: (i, 0)),
        out_shape=jax.ShapeDtypeStruct((NP, 64), jnp.bfloat16),
    )(h, deg2)

    h2raw = pl.pallas_call(
        _h2_body,
        grid=(KB, KB),
        in_specs=[
            pl.BlockSpec((BK, BK), lambda i, k: (i, k)),
            pl.BlockSpec((BK, 64), lambda i, k: (k, 0)),
        ],
        out_specs=pl.BlockSpec((BK, 64), lambda i, k: (i, 0)),
        out_shape=jax.ShapeDtypeStruct((NP, 64), jnp.float32),
        scratch_shapes=[pltpu.VMEM((BK, 64), jnp.float32)],
        compiler_params=pltpu.CompilerParams(
            dimension_semantics=("parallel", "arbitrary")),
    )(b2, hp2)

    out = pl.pallas_call(
        _final_body,
        grid=(KB,),
        in_specs=[
            pl.BlockSpec((BK, 64), lambda i: (i, 0)),
            pl.BlockSpec((BK, 64), lambda i: (i, 0)),
            pl.BlockSpec((BK, 64), lambda i: (i, 0)),
            pl.BlockSpec((BK, 128), lambda i: (i, 0)),
            pl.BlockSpec((BK, 128), lambda i: (i, 0)),
            pl.BlockSpec((192, 64), lambda i: (0, 0)),
        ],
        out_specs=pl.BlockSpec((BK, 64), lambda i: (i, 0)),
        out_shape=jax.ShapeDtypeStruct((NP, 64), jnp.float32),
    )(h, h1raw, h2raw, deg1, deg2, W_final)

    return out[:N]


# speedup vs baseline: 179.6369x; 3.0872x over previous
import functools

import jax
import jax.numpy as jnp
from jax import lax
from jax.experimental import pallas as pl
from jax.experimental.pallas import tpu as pltpu
from jax.experimental.pallas import tpu_sc as plsc

N = 10000
NP = 10240
E = 160000
NWORDS = NP * NP
PAD = 64
HALF = NWORDS // 2
Q = HALF // 16
ZB = 65536
CH = 2000
BK = 1024
KB = NP // BK


def _sc_build_body(src_hbm, dst_hbm, a_hbm, zbuf, sbuf, dbuf, ibuf, obuf, sem):
    c = lax.axis_index("c")
    s = lax.axis_index("s")
    zlo = c * HALF + s * Q

    zero16 = jnp.zeros((16,), jnp.float32)

    def fill_z(i, _):
        zbuf[pl.ds(i * 16, 16)] = zero16
        return 0

    lax.fori_loop(0, ZB // 16, fill_z, 0)

    lane = lax.iota(jnp.int32, 16)

    def zc(i, _):
        pltpu.sync_copy(zbuf, a_hbm.at[pl.ds(zlo + i * ZB, ZB)])
        return 0

    lax.fori_loop(0, Q // ZB, zc, 0)

    plsc.subcore_barrier()

    clo = c * HALF
    ebase = s * (E // 16)

    def chunk(ci, _):
        base = ebase + ci * CH
        pltpu.sync_copy(src_hbm.at[pl.ds(base, CH)], sbuf)
        pltpu.sync_copy(dst_hbm.at[pl.ds(base, CH)], dbuf)

        def step(t, _):
            sv = sbuf[pl.ds(t * 16, 16)]
            dv = dbuf[pl.ds(t * 16, 16)]
            flat = sv * NP + dv
            m = (flat >= clo) & (flat < clo + HALF) & (sv != dv)
            dummy = zlo + t * NP + N + lane * 8
            ibuf[pl.ds(t * 16, 16)] = jnp.where(m, flat, dummy)
            obuf[pl.ds(t * 16, 16)] = jnp.where(m, 1.0, 0.0)
            return 0

        lax.fori_loop(0, CH // 16, step, 0)
        pltpu.async_copy(obuf, a_hbm.at[ibuf], sem).wait()
        return 0

    lax.fori_loop(0, E // (16 * CH), chunk, 0)


@functools.cache
def _sc_build():
    return pl.kernel(
        _sc_build_body,
        out_type=jax.ShapeDtypeStruct((NWORDS + PAD,), jnp.float32),
        mesh=plsc.VectorSubcoreMesh(core_axis_name="c", subcore_axis_name="s"),
        scratch_types=[
            pltpu.VMEM((ZB,), jnp.float32),
            pltpu.VMEM((CH,), jnp.int32),
            pltpu.VMEM((CH,), jnp.int32),
            pltpu.VMEM((CH,), jnp.int32),
            pltpu.VMEM((CH,), jnp.float32),
            pltpu.SemaphoreType.DMA,
        ],
    )


def _dinv(col):
    return jnp.where(col > 0.0, 1.0 / jnp.sqrt(jnp.maximum(col, 1.0)), 0.0)


def _conv_body(a_ref, ab_ref, deg_ref):
    t = a_ref[...]
    ab_ref[...] = t.astype(jnp.bfloat16)
    rs = jnp.sum(t, axis=1)
    deg_ref[...] = jnp.broadcast_to(rs[:, None], deg_ref.shape)


def _h_body(x_ref, w1_ref, deg_ref, h_ref, hp_ref):
    h = jnp.maximum(jnp.dot(x_ref[...], w1_ref[...],
                            preferred_element_type=jnp.float32), 0.0)
    h_ref[...] = h
    d1 = _dinv(deg_ref[:, 0:1])
    hp_ref[...] = (d1 * h).astype(jnp.bfloat16)


def _mm_body(a_ik, a_kj, hp1_k, b_out, deg2_out, h1_out, acc):
    i = pl.program_id(0)
    j = pl.program_id(1)
    k = pl.program_id(2)
    kn = pl.num_programs(2)

    @pl.when(k == 0)
    def _():
        acc[...] = jnp.zeros_like(acc)

    acc[...] += jnp.dot(a_ik[...], a_kj[...],
                        preferred_element_type=jnp.float32)

    @pl.when(j == 0)
    def _():
        part = jnp.dot(a_ik[...], hp1_k[...],
                       preferred_element_type=jnp.float32)
        prev = jnp.where(k == 0, jnp.zeros_like(part), h1_out[...])
        h1_out[...] = prev + part

    @pl.when(k == kn - 1)
    def _():
        cval = acc[...]
        bt = (cval > 0.0).astype(jnp.float32)
        ri = lax.broadcasted_iota(jnp.int32, bt.shape, 0)
        ci = lax.broadcasted_iota(jnp.int32, bt.shape, 1)
        bt = jnp.where((ri == ci) & (i == j), 0.0, bt)
        b_out[...] = bt.astype(jnp.bfloat16)
        rs = jnp.sum(bt, axis=1)
        prev = jnp.where(j == 0, jnp.zeros_like(deg2_out[...]), deg2_out[...])
        deg2_out[...] = prev + jnp.broadcast_to(rs[:, None], deg2_out.shape)


def _hp2_body(h_ref, deg_ref, hp_ref):
    d2 = _dinv(deg_ref[:, 0:1])
    hp_ref[...] = (d2 * h_ref[...]).astype(jnp.bfloat16)


def _h2_body(b_ik, hp2_k, h2_out, acc):
    k = pl.program_id(1)
    kn = pl.num_programs(1)

    @pl.when(k == 0)
    def _():
        acc[...] = jnp.zeros_like(acc)

    acc[...] += jnp.dot(b_ik[...], hp2_k[...],
                        preferred_element_type=jnp.float32)

    @pl.when(k == kn - 1)
    def _():
        h2_out[...] = acc[...]


def _final_body(h_ref, h1_ref, h2_ref, d1_ref, d2_ref, wf_ref, o_ref):
    wf = wf_ref[...]
    d1 = _dinv(d1_ref[:, 0:1])
    d2 = _dinv(d2_ref[:, 0:1])
    out = jnp.dot(h_ref[...], wf[0:64], preferred_element_type=jnp.float32)
    out += jnp.dot(d1 * h1_ref[...], wf[64:128],
                   preferred_element_type=jnp.float32)
    out += jnp.dot(d2 * h2_ref[...], wf[128:192],
                   preferred_element_type=jnp.float32)
    o_ref[...] = out


def kernel(x, edge_index, W1, W_final):
    src = edge_index[0].astype(jnp.int32)
    dst = edge_index[1].astype(jnp.int32)

    a_flat = _sc_build()(src, dst)
    a2d = a_flat[:NWORDS].reshape(NP, NP)

    abf, deg1 = pl.pallas_call(
        _conv_body,
        grid=(NP // 128,),
        in_specs=[pl.BlockSpec((128, NP), lambda i: (i, 0))],
        out_specs=[
            pl.BlockSpec((128, NP), lambda i: (i, 0)),
            pl.BlockSpec((128, 128), lambda i: (i, 0)),
        ],
        out_shape=[
            jax.ShapeDtypeStruct((NP, NP), jnp.bfloat16),
            jax.ShapeDtypeStruct((NP, 128), jnp.float32),
        ],
    )(a2d)

    x_pad = jnp.zeros((NP, x.shape[1]), jnp.float32).at[:N].set(x)

    h, hp1 = pl.pallas_call(
        _h_body,
        grid=(KB,),
        in_specs=[
            pl.BlockSpec((BK, 128), lambda i: (i, 0)),
            pl.BlockSpec((128, 64), lambda i: (0, 0)),
            pl.BlockSpec((BK, 128), lambda i: (i, 0)),
        ],
        out_specs=[
            pl.BlockSpec((BK, 64), lambda i: (i, 0)),
            pl.BlockSpec((BK, 64), lambda i: (i, 0)),
        ],
        out_shape=[
            jax.ShapeDtypeStruct((NP, 64), jnp.float32),
            jax.ShapeDtypeStruct((NP, 64), jnp.bfloat16),
        ],
    )(x_pad, W1, deg1)

    b2, deg2, h1raw = pl.pallas_call(
        _mm_body,
        grid=(KB, KB, KB),
        in_specs=[
            pl.BlockSpec((BK, BK), lambda i, j, k: (i, k)),
            pl.BlockSpec((BK, BK), lambda i, j, k: (k, j)),
            pl.BlockSpec((BK, 64), lambda i, j, k: (k, 0)),
        ],
        out_specs=[
            pl.BlockSpec((BK, BK), lambda i, j, k: (i, j)),
            pl.BlockSpec((BK, 128), lambda i, j, k: (i, 0)),
            pl.BlockSpec((BK, 64), lambda i, j, k: (i, 0)),
        ],
        out_shape=[
            jax.ShapeDtypeStruct((NP, NP), jnp.bfloat16),
            jax.ShapeDtypeStruct((NP, 128), jnp.float32),
            jax.ShapeDtypeStruct((NP, 64), jnp.float32),
        ],
        scratch_shapes=[pltpu.VMEM((BK, BK), jnp.float32)],
        compiler_params=pltpu.CompilerParams(
            dimension_semantics=("parallel", "arbitrary", "arbitrary")),
    )(abf, abf, hp1)

    hp2 = pl.pallas_call(
        _hp2_body,
        grid=(KB,),
        in_specs=[
            pl.BlockSpec((BK, 64), lambda i: (i, 0)),
            pl.BlockSpec((BK, 128), lambda i: (i, 0)),
        ],
        out_specs=pl.BlockSpec((BK, 64), lambda i: (i, 0)),
        out_shape=jax.ShapeDtypeStruct((NP, 64), jnp.bfloat16),
    )(h, deg2)

    h2raw = pl.pallas_call(
        _h2_body,
        grid=(KB, KB),
        in_specs=[
            pl.BlockSpec((BK, BK), lambda i, k: (i, k)),
            pl.BlockSpec((BK, 64), lambda i, k: (k, 0)),
        ],
        out_specs=pl.BlockSpec((BK, 64), lambda i, k: (i, 0)),
        out_shape=jax.ShapeDtypeStruct((NP, 64), jnp.float32),
        scratch_shapes=[pltpu.VMEM((BK, 64), jnp.float32)],
        compiler_params=pltpu.CompilerParams(
            dimension_semantics=("parallel", "arbitrary")),
    )(b2, hp2)

    out = pl.pallas_call(
        _final_body,
        grid=(KB,),
        in_specs=[
            pl.BlockSpec((BK, 64), lambda i: (i, 0)),
            pl.BlockSpec((BK, 64), lambda i: (i, 0)),
            pl.BlockSpec((BK, 64), lambda i: (i, 0)),
            pl.BlockSpec((BK, 128), lambda i: (i, 0)),
            pl.BlockSpec((BK, 128), lambda i: (i, 0)),
            pl.BlockSpec((192, 64), lambda i: (0, 0)),
        ],
        out_specs=pl.BlockSpec((BK, 64), lambda i: (i, 0)),
        out_shape=jax.ShapeDtypeStruct((NP, 64), jnp.float32),
    )(h, h1raw, h2raw, deg1, deg2, W_final)

    return out[:N]


# speedup vs baseline: 183.1139x; 1.0194x over previous
import functools

import jax
import jax.numpy as jnp
from jax import lax
from jax.experimental import pallas as pl
from jax.experimental.pallas import tpu as pltpu
from jax.experimental.pallas import tpu_sc as plsc

N = 10000
NP = 10240
E = 160000
NWORDS = NP * NP
PAD = 64
HALF = NWORDS // 2
Q = HALF // 16
ZB = 65536
CH = 2000
BK = 1024
KB = NP // BK


def _sc_build_body(src_hbm, dst_hbm, a_hbm, zbuf, sbuf, dbuf, ibuf, obuf, sem):
    c = lax.axis_index("c")
    s = lax.axis_index("s")
    zlo = c * HALF + s * Q

    zero16 = jnp.zeros((16,), jnp.float32)

    def fill_z(i, _):
        zbuf[pl.ds(i * 16, 16)] = zero16
        return 0

    lax.fori_loop(0, ZB // 16, fill_z, 0)

    lane = lax.iota(jnp.int32, 16)

    def zc(i, _):
        pltpu.sync_copy(zbuf, a_hbm.at[pl.ds(zlo + i * ZB, ZB)])
        return 0

    lax.fori_loop(0, Q // ZB, zc, 0)

    plsc.subcore_barrier()

    clo = c * HALF
    ebase = s * (E // 16)

    def chunk(ci, _):
        base = ebase + ci * CH
        pltpu.sync_copy(src_hbm.at[pl.ds(base, CH)], sbuf)
        pltpu.sync_copy(dst_hbm.at[pl.ds(base, CH)], dbuf)

        def step(t, _):
            sv = sbuf[pl.ds(t * 16, 16)]
            dv = dbuf[pl.ds(t * 16, 16)]
            flat = sv * NP + dv
            m = (flat >= clo) & (flat < clo + HALF) & (sv != dv)
            dummy = zlo + t * NP + N + lane * 8
            ibuf[pl.ds(t * 16, 16)] = jnp.where(m, flat, dummy)
            obuf[pl.ds(t * 16, 16)] = jnp.where(m, 1.0, 0.0)
            return 0

        lax.fori_loop(0, CH // 16, step, 0)
        pltpu.async_copy(obuf, a_hbm.at[ibuf], sem).wait()
        return 0

    lax.fori_loop(0, E // (16 * CH), chunk, 0)


@functools.cache
def _sc_build():
    return pl.kernel(
        _sc_build_body,
        out_type=jax.ShapeDtypeStruct((NWORDS + PAD,), jnp.float32),
        mesh=plsc.VectorSubcoreMesh(core_axis_name="c", subcore_axis_name="s"),
        scratch_types=[
            pltpu.VMEM((ZB,), jnp.float32),
            pltpu.VMEM((CH,), jnp.int32),
            pltpu.VMEM((CH,), jnp.int32),
            pltpu.VMEM((CH,), jnp.int32),
            pltpu.VMEM((CH,), jnp.float32),
            pltpu.SemaphoreType.DMA,
        ],
    )


def _dinv(col):
    return jnp.where(col > 0.0, 1.0 / jnp.sqrt(jnp.maximum(col, 1.0)), 0.0)


def _conv_body(a_ref, ab_ref, deg_ref):
    t = a_ref[...]
    ab_ref[...] = t.astype(jnp.int8)
    rs = jnp.sum(t, axis=1)
    deg_ref[...] = jnp.broadcast_to(rs[:, None], deg_ref.shape)


def _h_body(x_ref, w1_ref, deg_ref, h_ref, hp_ref):
    h = jnp.maximum(jnp.dot(x_ref[...], w1_ref[...],
                            preferred_element_type=jnp.float32), 0.0)
    h_ref[...] = h
    d1 = _dinv(deg_ref[:, 0:1])
    hp_ref[...] = (d1 * h).astype(jnp.bfloat16)


def _mm_body(a_ik, a_kj, hp1_k, b_out, deg2_out, h1_out, acc):
    i = pl.program_id(0)
    j = pl.program_id(1)
    k = pl.program_id(2)
    kn = pl.num_programs(2)

    @pl.when(k == 0)
    def _():
        acc[...] = jnp.zeros_like(acc)

    acc[...] += jnp.dot(a_ik[...], a_kj[...],
                        preferred_element_type=jnp.int32)

    @pl.when(j == 0)
    def _():
        part = jnp.dot(a_ik[...].astype(jnp.bfloat16), hp1_k[...],
                       preferred_element_type=jnp.float32)
        prev = jnp.where(k == 0, jnp.zeros_like(part), h1_out[...])
        h1_out[...] = prev + part

    @pl.when(k == kn - 1)
    def _():
        cval = acc[...]
        bt = (cval > 0).astype(jnp.float32)
        ri = lax.broadcasted_iota(jnp.int32, bt.shape, 0)
        ci = lax.broadcasted_iota(jnp.int32, bt.shape, 1)
        bt = jnp.where((ri == ci) & (i == j), 0.0, bt)
        b_out[...] = bt.astype(jnp.int8)
        rs = jnp.sum(bt, axis=1)
        prev = jnp.where(j == 0, jnp.zeros_like(deg2_out[...]), deg2_out[...])
        deg2_out[...] = prev + jnp.broadcast_to(rs[:, None], deg2_out.shape)


def _hp2_body(h_ref, deg_ref, hp_ref):
    d2 = _dinv(deg_ref[:, 0:1])
    hp_ref[...] = (d2 * h_ref[...]).astype(jnp.bfloat16)


def _h2_body(b_ik, hp2_k, h2_out, acc):
    k = pl.program_id(1)
    kn = pl.num_programs(1)

    @pl.when(k == 0)
    def _():
        acc[...] = jnp.zeros_like(acc)

    acc[...] += jnp.dot(b_ik[...].astype(jnp.bfloat16), hp2_k[...],
                        preferred_element_type=jnp.float32)

    @pl.when(k == kn - 1)
    def _():
        h2_out[...] = acc[...]


def _final_body(h_ref, h1_ref, h2_ref, d1_ref, d2_ref, wf_ref, o_ref):
    wf = wf_ref[...]
    d1 = _dinv(d1_ref[:, 0:1])
    d2 = _dinv(d2_ref[:, 0:1])
    out = jnp.dot(h_ref[...], wf[0:64], preferred_element_type=jnp.float32)
    out += jnp.dot(d1 * h1_ref[...], wf[64:128],
                   preferred_element_type=jnp.float32)
    out += jnp.dot(d2 * h2_ref[...], wf[128:192],
                   preferred_element_type=jnp.float32)
    o_ref[...] = out


def kernel(x, edge_index, W1, W_final):
    src = edge_index[0].astype(jnp.int32)
    dst = edge_index[1].astype(jnp.int32)

    a_flat = _sc_build()(src, dst)
    a2d = a_flat[:NWORDS].reshape(NP, NP)

    abf, deg1 = pl.pallas_call(
        _conv_body,
        grid=(NP // 128,),
        in_specs=[pl.BlockSpec((128, NP), lambda i: (i, 0))],
        out_specs=[
            pl.BlockSpec((128, NP), lambda i: (i, 0)),
            pl.BlockSpec((128, 128), lambda i: (i, 0)),
        ],
        out_shape=[
            jax.ShapeDtypeStruct((NP, NP), jnp.int8),
            jax.ShapeDtypeStruct((NP, 128), jnp.float32),
        ],
    )(a2d)

    x_pad = jnp.zeros((NP, x.shape[1]), jnp.float32).at[:N].set(x)

    h, hp1 = pl.pallas_call(
        _h_body,
        grid=(KB,),
        in_specs=[
            pl.BlockSpec((BK, 128), lambda i: (i, 0)),
            pl.BlockSpec((128, 64), lambda i: (0, 0)),
            pl.BlockSpec((BK, 128), lambda i: (i, 0)),
        ],
        out_specs=[
            pl.BlockSpec((BK, 64), lambda i: (i, 0)),
            pl.BlockSpec((BK, 64), lambda i: (i, 0)),
        ],
        out_shape=[
            jax.ShapeDtypeStruct((NP, 64), jnp.float32),
            jax.ShapeDtypeStruct((NP, 64), jnp.bfloat16),
        ],
    )(x_pad, W1, deg1)

    b2, deg2, h1raw = pl.pallas_call(
        _mm_body,
        grid=(KB, KB, KB),
        in_specs=[
            pl.BlockSpec((BK, BK), lambda i, j, k: (i, k)),
            pl.BlockSpec((BK, BK), lambda i, j, k: (k, j)),
            pl.BlockSpec((BK, 64), lambda i, j, k: (k, 0)),
        ],
        out_specs=[
            pl.BlockSpec((BK, BK), lambda i, j, k: (i, j)),
            pl.BlockSpec((BK, 128), lambda i, j, k: (i, 0)),
            pl.BlockSpec((BK, 64), lambda i, j, k: (i, 0)),
        ],
        out_shape=[
            jax.ShapeDtypeStruct((NP, NP), jnp.int8),
            jax.ShapeDtypeStruct((NP, 128), jnp.float32),
            jax.ShapeDtypeStruct((NP, 64), jnp.float32),
        ],
        scratch_shapes=[pltpu.VMEM((BK, BK), jnp.int32)],
        compiler_params=pltpu.CompilerParams(
            dimension_semantics=("parallel", "arbitrary", "arbitrary")),
    )(abf, abf, hp1)

    hp2 = pl.pallas_call(
        _hp2_body,
        grid=(KB,),
        in_specs=[
            pl.BlockSpec((BK, 64), lambda i: (i, 0)),
            pl.BlockSpec((BK, 128), lambda i: (i, 0)),
        ],
        out_specs=pl.BlockSpec((BK, 64), lambda i: (i, 0)),
        out_shape=jax.ShapeDtypeStruct((NP, 64), jnp.bfloat16),
    )(h, deg2)

    h2raw = pl.pallas_call(
        _h2_body,
        grid=(KB, KB),
        in_specs=[
            pl.BlockSpec((BK, BK), lambda i, k: (i, k)),
            pl.BlockSpec((BK, 64), lambda i, k: (k, 0)),
        ],
        out_specs=pl.BlockSpec((BK, 64), lambda i, k: (i, 0)),
        out_shape=jax.ShapeDtypeStruct((NP, 64), jnp.float32),
        scratch_shapes=[pltpu.VMEM((BK, 64), jnp.float32)],
        compiler_params=pltpu.CompilerParams(
            dimension_semantics=("parallel", "arbitrary")),
    )(b2, hp2)

    out = pl.pallas_call(
        _final_body,
        grid=(KB,),
        in_specs=[
            pl.BlockSpec((BK, 64), lambda i: (i, 0)),
            pl.BlockSpec((BK, 64), lambda i: (i, 0)),
            pl.BlockSpec((BK, 64), lambda i: (i, 0)),
            pl.BlockSpec((BK, 128), lambda i: (i, 0)),
            pl.BlockSpec((BK, 128), lambda i: (i, 0)),
            pl.BlockSpec((192, 64), lambda i: (0, 0)),
        ],
        out_specs=pl.BlockSpec((BK, 64), lambda i: (i, 0)),
        out_shape=jax.ShapeDtypeStruct((NP, 64), jnp.float32),
    )(h, h1raw, h2raw, deg1, deg2, W_final)

    return out[:N]


# speedup vs baseline: 250.9046x; 1.3702x over previous
import functools

import jax
import jax.numpy as jnp
from jax import lax
from jax.experimental import pallas as pl
from jax.experimental.pallas import tpu as pltpu
from jax.experimental.pallas import tpu_sc as plsc

N = 10000
NP = 10240
E = 160000
NWORDS = NP * NP
PAD = 64
HALF = NWORDS // 2
Q = HALF // 16
ZB = 65536
CH = 2000
BK = 1024
KB = NP // BK
BKK = 2048
KBK = NP // BKK


def _sc_build_body(src_hbm, dst_hbm, a_hbm, zbuf, sbuf, dbuf, ibuf, obuf, sem):
    c = lax.axis_index("c")
    s = lax.axis_index("s")
    zlo = c * HALF + s * Q

    zero16 = jnp.zeros((16,), jnp.float32)

    def fill_z(i, _):
        zbuf[pl.ds(i * 16, 16)] = zero16
        return 0

    lax.fori_loop(0, ZB // 16, fill_z, 0)

    lane = lax.iota(jnp.int32, 16)

    def zc(i, _):
        pltpu.sync_copy(zbuf, a_hbm.at[pl.ds(zlo + i * ZB, ZB)])
        return 0

    lax.fori_loop(0, Q // ZB, zc, 0)

    plsc.subcore_barrier()

    clo = c * HALF
    ebase = s * (E // 16)

    def chunk(ci, _):
        base = ebase + ci * CH
        pltpu.sync_copy(src_hbm.at[pl.ds(base, CH)], sbuf)
        pltpu.sync_copy(dst_hbm.at[pl.ds(base, CH)], dbuf)

        def step(t, _):
            sv = sbuf[pl.ds(t * 16, 16)]
            dv = dbuf[pl.ds(t * 16, 16)]
            flat = sv * NP + dv
            m = (flat >= clo) & (flat < clo + HALF) & (sv != dv)
            dummy = zlo + t * NP + N + lane * 8
            ibuf[pl.ds(t * 16, 16)] = jnp.where(m, flat, dummy)
            obuf[pl.ds(t * 16, 16)] = jnp.where(m, 1.0, 0.0)
            return 0

        lax.fori_loop(0, CH // 16, step, 0)
        pltpu.async_copy(obuf, a_hbm.at[ibuf], sem).wait()
        return 0

    lax.fori_loop(0, E // (16 * CH), chunk, 0)


@functools.cache
def _sc_build():
    return pl.kernel(
        _sc_build_body,
        out_type=jax.ShapeDtypeStruct((NWORDS + PAD,), jnp.float32),
        mesh=plsc.VectorSubcoreMesh(core_axis_name="c", subcore_axis_name="s"),
        scratch_types=[
            pltpu.VMEM((ZB,), jnp.float32),
            pltpu.VMEM((CH,), jnp.int32),
            pltpu.VMEM((CH,), jnp.int32),
            pltpu.VMEM((CH,), jnp.int32),
            pltpu.VMEM((CH,), jnp.float32),
            pltpu.SemaphoreType.DMA,
        ],
    )


def _dinv(col):
    return jnp.where(col > 0.0, 1.0 / jnp.sqrt(jnp.maximum(col, 1.0)), 0.0)


def _conv_body(a_ref, ab_ref, deg_ref):
    t = a_ref[...]
    ab_ref[...] = t.astype(jnp.int4)
    rs = jnp.sum(t, axis=1)
    deg_ref[...] = jnp.broadcast_to(rs[:, None], deg_ref.shape)


def _h_body(x_ref, w1_ref, deg_ref, h_ref, hp_ref):
    h = jnp.maximum(jnp.dot(x_ref[...], w1_ref[...],
                            preferred_element_type=jnp.float32), 0.0)
    h_ref[...] = h
    d1 = _dinv(deg_ref[:, 0:1])
    hp_ref[...] = (d1 * h).astype(jnp.bfloat16)


def _mm_body(a_ik, a_kj, hp1_k, b_out, deg2_out, h1_out, acc):
    i = pl.program_id(0)
    j = pl.program_id(1)
    k = pl.program_id(2)
    kn = pl.num_programs(2)

    part = jnp.dot(a_ik[...], a_kj[...], preferred_element_type=jnp.int32)

    @pl.when(k == 0)
    def _():
        acc[...] = part

    @pl.when(k != 0)
    def _():
        acc[...] += part

    @pl.when(j == 0)
    def _():
        part = jnp.dot(a_ik[...].astype(jnp.bfloat16), hp1_k[...],
                       preferred_element_type=jnp.float32)
        prev = jnp.where(k == 0, jnp.zeros_like(part), h1_out[...])
        h1_out[...] = prev + part

    @pl.when(k == kn - 1)
    def _():
        cval = acc[...]
        bt = (cval > 0).astype(jnp.float32)
        ri = lax.broadcasted_iota(jnp.int32, bt.shape, 0)
        ci = lax.broadcasted_iota(jnp.int32, bt.shape, 1)
        bt = jnp.where((ri == ci) & (i == j), 0.0, bt)
        b_out[...] = bt.astype(jnp.int8)
        rs = jnp.sum(bt, axis=1)
        prev = jnp.where(j == 0, jnp.zeros_like(deg2_out[...]), deg2_out[...])
        deg2_out[...] = prev + jnp.broadcast_to(rs[:, None], deg2_out.shape)


def _hp2_body(h_ref, deg_ref, hp_ref):
    d2 = _dinv(deg_ref[:, 0:1])
    hp_ref[...] = (d2 * h_ref[...]).astype(jnp.bfloat16)


def _h2_body(b_ik, hp2_k, h2_out, acc):
    k = pl.program_id(1)
    kn = pl.num_programs(1)

    @pl.when(k == 0)
    def _():
        acc[...] = jnp.zeros_like(acc)

    acc[...] += jnp.dot(b_ik[...].astype(jnp.bfloat16), hp2_k[...],
                        preferred_element_type=jnp.float32)

    @pl.when(k == kn - 1)
    def _():
        h2_out[...] = acc[...]


def _final_body(h_ref, h1_ref, h2_ref, d1_ref, d2_ref, wf_ref, o_ref):
    wf = wf_ref[...]
    d1 = _dinv(d1_ref[:, 0:1])
    d2 = _dinv(d2_ref[:, 0:1])
    out = jnp.dot(h_ref[...], wf[0:64], preferred_element_type=jnp.float32)
    out += jnp.dot(d1 * h1_ref[...], wf[64:128],
                   preferred_element_type=jnp.float32)
    out += jnp.dot(d2 * h2_ref[...], wf[128:192],
                   preferred_element_type=jnp.float32)
    o_ref[...] = out


def kernel(x, edge_index, W1, W_final):
    src = edge_index[0].astype(jnp.int32)
    dst = edge_index[1].astype(jnp.int32)

    a_flat = _sc_build()(src, dst)
    a2d = a_flat[:NWORDS].reshape(NP, NP)

    abf, deg1 = pl.pallas_call(
        _conv_body,
        grid=(NP // 128,),
        in_specs=[pl.BlockSpec((128, NP), lambda i: (i, 0))],
        out_specs=[
            pl.BlockSpec((128, NP), lambda i: (i, 0)),
            pl.BlockSpec((128, 128), lambda i: (i, 0)),
        ],
        out_shape=[
            jax.ShapeDtypeStruct((NP, NP), jnp.int4),
            jax.ShapeDtypeStruct((NP, 128), jnp.float32),
        ],
    )(a2d)

    x_pad = jnp.zeros((NP, x.shape[1]), jnp.float32).at[:N].set(x)

    h, hp1 = pl.pallas_call(
        _h_body,
        grid=(KB,),
        in_specs=[
            pl.BlockSpec((BK, 128), lambda i: (i, 0)),
            pl.BlockSpec((128, 64), lambda i: (0, 0)),
            pl.BlockSpec((BK, 128), lambda i: (i, 0)),
        ],
        out_specs=[
            pl.BlockSpec((BK, 64), lambda i: (i, 0)),
            pl.BlockSpec((BK, 64), lambda i: (i, 0)),
        ],
        out_shape=[
            jax.ShapeDtypeStruct((NP, 64), jnp.float32),
            jax.ShapeDtypeStruct((NP, 64), jnp.bfloat16),
        ],
    )(x_pad, W1, deg1)

    b2, deg2, h1raw = pl.pallas_call(
        _mm_body,
        grid=(KB, KB, KBK),
        in_specs=[
            pl.BlockSpec((BK, BKK), lambda i, j, k: (i, k)),
            pl.BlockSpec((BKK, BK), lambda i, j, k: (k, j)),
            pl.BlockSpec((BKK, 64), lambda i, j, k: (k, 0)),
        ],
        out_specs=[
            pl.BlockSpec((BK, BK), lambda i, j, k: (i, j)),
            pl.BlockSpec((BK, 128), lambda i, j, k: (i, 0)),
            pl.BlockSpec((BK, 64), lambda i, j, k: (i, 0)),
        ],
        out_shape=[
            jax.ShapeDtypeStruct((NP, NP), jnp.int8),
            jax.ShapeDtypeStruct((NP, 128), jnp.float32),
            jax.ShapeDtypeStruct((NP, 64), jnp.float32),
        ],
        scratch_shapes=[pltpu.VMEM((BK, BK), jnp.int32)],
        compiler_params=pltpu.CompilerParams(
            dimension_semantics=("parallel", "arbitrary", "arbitrary")),
    )(abf, abf, hp1)

    hp2 = pl.pallas_call(
        _hp2_body,
        grid=(KB,),
        in_specs=[
            pl.BlockSpec((BK, 64), lambda i: (i, 0)),
            pl.BlockSpec((BK, 128), lambda i: (i, 0)),
        ],
        out_specs=pl.BlockSpec((BK, 64), lambda i: (i, 0)),
        out_shape=jax.ShapeDtypeStruct((NP, 64), jnp.bfloat16),
    )(h, deg2)

    h2raw = pl.pallas_call(
        _h2_body,
        grid=(KB, KB),
        in_specs=[
            pl.BlockSpec((BK, BK), lambda i, k: (i, k)),
            pl.BlockSpec((BK, 64), lambda i, k: (k, 0)),
        ],
        out_specs=pl.BlockSpec((BK, 64), lambda i, k: (i, 0)),
        out_shape=jax.ShapeDtypeStruct((NP, 64), jnp.float32),
        scratch_shapes=[pltpu.VMEM((BK, 64), jnp.float32)],
        compiler_params=pltpu.CompilerParams(
            dimension_semantics=("parallel", "arbitrary")),
    )(b2, hp2)

    out = pl.pallas_call(
        _final_body,
        grid=(KB,),
        in_specs=[
            pl.BlockSpec((BK, 64), lambda i: (i, 0)),
            pl.BlockSpec((BK, 64), lambda i: (i, 0)),
            pl.BlockSpec((BK, 64), lambda i: (i, 0)),
            pl.BlockSpec((BK, 128), lambda i: (i, 0)),
            pl.BlockSpec((BK, 128), lambda i: (i, 0)),
            pl.BlockSpec((192, 64), lambda i: (0, 0)),
        ],
        out_specs=pl.BlockSpec((BK, 64), lambda i: (i, 0)),
        out_shape=jax.ShapeDtypeStruct((NP, 64), jnp.float32),
    )(h, h1raw, h2raw, deg1, deg2, W_final)

    return out[:N]


# speedup vs baseline: 282.9927x; 1.1279x over previous
import functools

import jax
import jax.numpy as jnp
from jax import lax
from jax.experimental import pallas as pl
from jax.experimental.pallas import tpu as pltpu
from jax.experimental.pallas import tpu_sc as plsc

N = 10000
NP = 10240
E = 160000
NWORDS = NP * NP
PAD = 64
HALF = NWORDS // 2
Q = HALF // 16
ZB = 65536
CH = 2000
BK = 1024
KB = NP // BK
BKK = 2048
KBK = NP // BKK


def _sc_build_body(src_hbm, dst_hbm, a_hbm, zbuf, sbuf, dbuf, ibuf, obuf, sem):
    c = lax.axis_index("c")
    s = lax.axis_index("s")
    zlo = c * HALF + s * Q

    zero16 = jnp.zeros((16,), jnp.float32)

    def fill_z(i, _):
        zbuf[pl.ds(i * 16, 16)] = zero16
        return 0

    lax.fori_loop(0, ZB // 16, fill_z, 0)

    lane = lax.iota(jnp.int32, 16)

    def zc(i, _):
        pltpu.sync_copy(zbuf, a_hbm.at[pl.ds(zlo + i * ZB, ZB)])
        return 0

    lax.fori_loop(0, Q // ZB, zc, 0)

    plsc.subcore_barrier()

    clo = c * HALF
    ebase = s * (E // 16)

    def chunk(ci, _):
        base = ebase + ci * CH
        pltpu.sync_copy(src_hbm.at[pl.ds(base, CH)], sbuf)
        pltpu.sync_copy(dst_hbm.at[pl.ds(base, CH)], dbuf)

        def step(t, _):
            sv = sbuf[pl.ds(t * 16, 16)]
            dv = dbuf[pl.ds(t * 16, 16)]
            flat = sv * NP + dv
            m = (flat >= clo) & (flat < clo + HALF) & (sv != dv)
            dummy = zlo + t * NP + N + lane * 8
            ibuf[pl.ds(t * 16, 16)] = jnp.where(m, flat, dummy)
            obuf[pl.ds(t * 16, 16)] = jnp.where(m, 1.0, 0.0)
            return 0

        lax.fori_loop(0, CH // 16, step, 0)
        pltpu.async_copy(obuf, a_hbm.at[ibuf], sem).wait()
        return 0

    lax.fori_loop(0, E // (16 * CH), chunk, 0)


@functools.cache
def _sc_build():
    return pl.kernel(
        _sc_build_body,
        out_type=jax.ShapeDtypeStruct((NWORDS + PAD,), jnp.float32),
        mesh=plsc.VectorSubcoreMesh(core_axis_name="c", subcore_axis_name="s"),
        scratch_types=[
            pltpu.VMEM((ZB,), jnp.float32),
            pltpu.VMEM((CH,), jnp.int32),
            pltpu.VMEM((CH,), jnp.int32),
            pltpu.VMEM((CH,), jnp.int32),
            pltpu.VMEM((CH,), jnp.float32),
            pltpu.SemaphoreType.DMA,
        ],
    )


def _dinv(col):
    return jnp.where(col > 0.0, 1.0 / jnp.sqrt(jnp.maximum(col, 1.0)), 0.0)


def _conv_body(a_ref, ab_ref, deg_ref):
    t = a_ref[...]
    ab_ref[...] = t.astype(jnp.int4)
    rs = jnp.sum(t, axis=1)
    deg_ref[...] = jnp.broadcast_to(rs[:, None], deg_ref.shape)


def _h_body(x_ref, w1_ref, deg_ref, h_ref, hp_ref):
    h = jnp.maximum(jnp.dot(x_ref[...], w1_ref[...],
                            preferred_element_type=jnp.float32), 0.0)
    h_ref[...] = h
    d1 = _dinv(deg_ref[:, 0:1])
    hp_ref[...] = (d1 * h).astype(jnp.bfloat16)


def _mm_body(a_ik, a_kj, hp1_k, b_out, deg2_out, h1_out):
    i = pl.program_id(0)
    j = pl.program_id(1)

    cval = jnp.dot(a_ik[...], a_kj[...], preferred_element_type=jnp.int32)
    bt = (cval > 0).astype(jnp.float32)
    ri = lax.broadcasted_iota(jnp.int32, bt.shape, 0)
    ci = lax.broadcasted_iota(jnp.int32, bt.shape, 1)
    bt = jnp.where((ri == ci) & (i == j), 0.0, bt)
    b_out[...] = bt.astype(jnp.int8)
    rs = jnp.sum(bt, axis=1)
    prev = jnp.where(j == 0, jnp.zeros_like(deg2_out[...]), deg2_out[...])
    deg2_out[...] = prev + jnp.broadcast_to(rs[:, None], deg2_out.shape)

    @pl.when(j == 0)
    def _():
        h1_out[...] = jnp.dot(a_ik[...].astype(jnp.bfloat16), hp1_k[...],
                              preferred_element_type=jnp.float32)


def _hp2_body(h_ref, deg_ref, hp_ref):
    d2 = _dinv(deg_ref[:, 0:1])
    hp_ref[...] = (d2 * h_ref[...]).astype(jnp.bfloat16)


def _h2_body(b_ik, hp2_k, h2_out):
    h2_out[...] = jnp.dot(b_ik[...].astype(jnp.bfloat16), hp2_k[...],
                          preferred_element_type=jnp.float32)


def _final_body(h_ref, h1_ref, h2_ref, d1_ref, d2_ref, wf_ref, o_ref):
    wf = wf_ref[...]
    d1 = _dinv(d1_ref[:, 0:1])
    d2 = _dinv(d2_ref[:, 0:1])
    out = jnp.dot(h_ref[...], wf[0:64], preferred_element_type=jnp.float32)
    out += jnp.dot(d1 * h1_ref[...], wf[64:128],
                   preferred_element_type=jnp.float32)
    out += jnp.dot(d2 * h2_ref[...], wf[128:192],
                   preferred_element_type=jnp.float32)
    o_ref[...] = out


def kernel(x, edge_index, W1, W_final):
    src = edge_index[0].astype(jnp.int32)
    dst = edge_index[1].astype(jnp.int32)

    a_flat = _sc_build()(src, dst)
    a2d = a_flat[:NWORDS].reshape(NP, NP)

    abf, deg1 = pl.pallas_call(
        _conv_body,
        grid=(NP // 128,),
        in_specs=[pl.BlockSpec((128, NP), lambda i: (i, 0))],
        out_specs=[
            pl.BlockSpec((128, NP), lambda i: (i, 0)),
            pl.BlockSpec((128, 128), lambda i: (i, 0)),
        ],
        out_shape=[
            jax.ShapeDtypeStruct((NP, NP), jnp.int4),
            jax.ShapeDtypeStruct((NP, 128), jnp.float32),
        ],
    )(a2d)

    x_pad = jnp.zeros((NP, x.shape[1]), jnp.float32).at[:N].set(x)

    h, hp1 = pl.pallas_call(
        _h_body,
        grid=(KB,),
        in_specs=[
            pl.BlockSpec((BK, 128), lambda i: (i, 0)),
            pl.BlockSpec((128, 64), lambda i: (0, 0)),
            pl.BlockSpec((BK, 128), lambda i: (i, 0)),
        ],
        out_specs=[
            pl.BlockSpec((BK, 64), lambda i: (i, 0)),
            pl.BlockSpec((BK, 64), lambda i: (i, 0)),
        ],
        out_shape=[
            jax.ShapeDtypeStruct((NP, 64), jnp.float32),
            jax.ShapeDtypeStruct((NP, 64), jnp.bfloat16),
        ],
    )(x_pad, W1, deg1)

    b2, deg2, h1raw = pl.pallas_call(
        _mm_body,
        grid=(KB, KB),
        in_specs=[
            pl.BlockSpec((BK, NP), lambda i, j: (i, 0)),
            pl.BlockSpec((NP, BK), lambda i, j: (0, j)),
            pl.BlockSpec((NP, 64), lambda i, j: (0, 0)),
        ],
        out_specs=[
            pl.BlockSpec((BK, BK), lambda i, j: (i, j)),
            pl.BlockSpec((BK, 128), lambda i, j: (i, 0)),
            pl.BlockSpec((BK, 64), lambda i, j: (i, 0)),
        ],
        out_shape=[
            jax.ShapeDtypeStruct((NP, NP), jnp.int8),
            jax.ShapeDtypeStruct((NP, 128), jnp.float32),
            jax.ShapeDtypeStruct((NP, 64), jnp.float32),
        ],
        compiler_params=pltpu.CompilerParams(
            dimension_semantics=("parallel", "arbitrary")),
    )(abf, abf, hp1)

    hp2 = pl.pallas_call(
        _hp2_body,
        grid=(KB,),
        in_specs=[
            pl.BlockSpec((BK, 64), lambda i: (i, 0)),
            pl.BlockSpec((BK, 128), lambda i: (i, 0)),
        ],
        out_specs=pl.BlockSpec((BK, 64), lambda i: (i, 0)),
        out_shape=jax.ShapeDtypeStruct((NP, 64), jnp.bfloat16),
    )(h, deg2)

    h2raw = pl.pallas_call(
        _h2_body,
        grid=(KB,),
        in_specs=[
            pl.BlockSpec((BK, NP), lambda i: (i, 0)),
            pl.BlockSpec((NP, 64), lambda i: (0, 0)),
        ],
        out_specs=pl.BlockSpec((BK, 64), lambda i: (i, 0)),
        out_shape=jax.ShapeDtypeStruct((NP, 64), jnp.float32),
    )(b2, hp2)

    out = pl.pallas_call(
        _final_body,
        grid=(KB,),
        in_specs=[
            pl.BlockSpec((BK, 64), lambda i: (i, 0)),
            pl.BlockSpec((BK, 64), lambda i: (i, 0)),
            pl.BlockSpec((BK, 64), lambda i: (i, 0)),
            pl.BlockSpec((BK, 128), lambda i: (i, 0)),
            pl.BlockSpec((BK, 128), lambda i: (i, 0)),
            pl.BlockSpec((192, 64), lambda i: (0, 0)),
        ],
        out_specs=pl.BlockSpec((BK, 64), lambda i: (i, 0)),
        out_shape=jax.ShapeDtypeStruct((NP, 64), jnp.float32),
    )(h, h1raw, h2raw, deg1, deg2, W_final)

    return out[:N]


# speedup vs baseline: 283.1371x; 1.0005x over previous
import functools

import jax
import jax.numpy as jnp
from jax import lax
from jax.experimental import pallas as pl
from jax.experimental.pallas import tpu as pltpu
from jax.experimental.pallas import tpu_sc as plsc

N = 10000
NP = 10240
E = 160000
NWORDS = NP * NP
PAD = 64
HALF = NWORDS // 2
Q = HALF // 16
ZB = 65536
CH = 2000
BK = 1024
KB = NP // BK
BKK = 2048
KBK = NP // BKK


def _sc_build_body(src_hbm, dst_hbm, a_hbm, zbuf, sbuf, dbuf,
                   i0, i1, i2, i3, i4, o0, o1, o2, o3, o4, sem, zsem):
    ibufs = (i0, i1, i2, i3, i4)
    obufs = (o0, o1, o2, o3, o4)
    c = lax.axis_index("c")
    s = lax.axis_index("s")
    zlo = c * HALF + s * Q

    zero16 = jnp.zeros((16,), jnp.float32)

    def fill_z(i, _):
        zbuf[pl.ds(i * 16, 16)] = zero16
        return 0

    lax.fori_loop(0, ZB // 16, fill_z, 0)

    lane = lax.iota(jnp.int32, 16)

    def zc(i, _):
        pltpu.async_copy(zbuf, a_hbm.at[pl.ds(zlo + i * ZB, ZB)], zsem)
        return 0

    lax.fori_loop(0, Q // ZB, zc, 0)

    clo = c * HALF
    ebase = s * (E // 16)

    for ci in range(E // (16 * CH)):
        base = ebase + ci * CH
        pltpu.sync_copy(src_hbm.at[pl.ds(base, CH)], sbuf)
        pltpu.sync_copy(dst_hbm.at[pl.ds(base, CH)], dbuf)
        ibuf = ibufs[ci]
        obuf = obufs[ci]

        def step(t, _, ibuf=ibuf, obuf=obuf):
            sv = sbuf[pl.ds(t * 16, 16)]
            dv = dbuf[pl.ds(t * 16, 16)]
            flat = sv * NP + dv
            m = (flat >= clo) & (flat < clo + HALF) & (sv != dv)
            dummy = zlo + t * NP + N + lane * 8
            ibuf[pl.ds(t * 16, 16)] = jnp.where(m, flat, dummy)
            obuf[pl.ds(t * 16, 16)] = jnp.where(m, 1.0, 0.0)
            return 0

        lax.fori_loop(0, CH // 16, step, 0)

    def zdrain(i, _):
        pltpu.make_async_copy(zbuf, a_hbm.at[pl.ds(zlo + i * ZB, ZB)],
                              zsem).wait()
        return 0

    lax.fori_loop(0, Q // ZB, zdrain, 0)
    plsc.subcore_barrier()

    for ci in range(E // (16 * CH)):
        pltpu.async_copy(obufs[ci], a_hbm.at[ibufs[ci]], sem)
    for ci in range(E // (16 * CH)):
        pltpu.make_async_copy(obufs[ci], a_hbm.at[ibufs[ci]], sem).wait()


@functools.cache
def _sc_build():
    return pl.kernel(
        _sc_build_body,
        out_type=jax.ShapeDtypeStruct((NWORDS + PAD,), jnp.float32),
        mesh=plsc.VectorSubcoreMesh(core_axis_name="c", subcore_axis_name="s"),
        scratch_types=[
            pltpu.VMEM((ZB,), jnp.float32),
            pltpu.VMEM((CH,), jnp.int32),
            pltpu.VMEM((CH,), jnp.int32),
            *[pltpu.VMEM((CH,), jnp.int32) for _ in range(5)],
            *[pltpu.VMEM((CH,), jnp.float32) for _ in range(5)],
            pltpu.SemaphoreType.DMA,
            pltpu.SemaphoreType.DMA,
        ],
    )


def _dinv(col):
    return jnp.where(col > 0.0, 1.0 / jnp.sqrt(jnp.maximum(col, 1.0)), 0.0)


def _conv_body(a_ref, ab_ref, deg_ref):
    t = a_ref[...]
    ab_ref[...] = t.astype(jnp.int4)
    rs = jnp.sum(t, axis=1)
    deg_ref[...] = jnp.broadcast_to(rs[:, None], deg_ref.shape)


def _h_body(x_ref, w1_ref, deg_ref, h_ref, hp_ref):
    h = jnp.maximum(jnp.dot(x_ref[...], w1_ref[...],
                            preferred_element_type=jnp.float32), 0.0)
    h_ref[...] = h
    d1 = _dinv(deg_ref[:, 0:1])
    hp_ref[...] = (d1 * h).astype(jnp.bfloat16)


def _mm_body(a_ik, a_kj, hp1_k, b_out, deg2_out, h1_out):
    i = pl.program_id(0)
    j = pl.program_id(1)

    cval = jnp.dot(a_ik[...], a_kj[...], preferred_element_type=jnp.int32)
    bt = (cval > 0).astype(jnp.float32)
    ri = lax.broadcasted_iota(jnp.int32, bt.shape, 0)
    ci = lax.broadcasted_iota(jnp.int32, bt.shape, 1)
    bt = jnp.where((ri == ci) & (i == j), 0.0, bt)
    b_out[...] = bt.astype(jnp.int8)
    rs = jnp.sum(bt, axis=1)
    prev = jnp.where(j == 0, jnp.zeros_like(deg2_out[...]), deg2_out[...])
    deg2_out[...] = prev + jnp.broadcast_to(rs[:, None], deg2_out.shape)

    @pl.when(j == 0)
    def _():
        h1_out[...] = jnp.dot(a_ik[...].astype(jnp.bfloat16), hp1_k[...],
                              preferred_element_type=jnp.float32)


def _hp2_body(h_ref, deg_ref, hp_ref):
    d2 = _dinv(deg_ref[:, 0:1])
    hp_ref[...] = (d2 * h_ref[...]).astype(jnp.bfloat16)


def _h2_body(b_ik, hp2_k, h2_out):
    h2_out[...] = jnp.dot(b_ik[...].astype(jnp.bfloat16), hp2_k[...],
                          preferred_element_type=jnp.float32)


def _final_body(h_ref, h1_ref, h2_ref, d1_ref, d2_ref, wf_ref, o_ref):
    wf = wf_ref[...]
    d1 = _dinv(d1_ref[:, 0:1])
    d2 = _dinv(d2_ref[:, 0:1])
    out = jnp.dot(h_ref[...], wf[0:64], preferred_element_type=jnp.float32)
    out += jnp.dot(d1 * h1_ref[...], wf[64:128],
                   preferred_element_type=jnp.float32)
    out += jnp.dot(d2 * h2_ref[...], wf[128:192],
                   preferred_element_type=jnp.float32)
    o_ref[...] = out


def kernel(x, edge_index, W1, W_final):
    src = edge_index[0].astype(jnp.int32)
    dst = edge_index[1].astype(jnp.int32)

    a_flat = _sc_build()(src, dst)
    a2d = a_flat[:NWORDS].reshape(NP, NP)

    abf, deg1 = pl.pallas_call(
        _conv_body,
        grid=(NP // 128,),
        in_specs=[pl.BlockSpec((128, NP), lambda i: (i, 0))],
        out_specs=[
            pl.BlockSpec((128, NP), lambda i: (i, 0)),
            pl.BlockSpec((128, 128), lambda i: (i, 0)),
        ],
        out_shape=[
            jax.ShapeDtypeStruct((NP, NP), jnp.int4),
            jax.ShapeDtypeStruct((NP, 128), jnp.float32),
        ],
    )(a2d)

    x_pad = jnp.zeros((NP, x.shape[1]), jnp.float32).at[:N].set(x)

    h, hp1 = pl.pallas_call(
        _h_body,
        grid=(KB,),
        in_specs=[
            pl.BlockSpec((BK, 128), lambda i: (i, 0)),
            pl.BlockSpec((128, 64), lambda i: (0, 0)),
            pl.BlockSpec((BK, 128), lambda i: (i, 0)),
        ],
        out_specs=[
            pl.BlockSpec((BK, 64), lambda i: (i, 0)),
            pl.BlockSpec((BK, 64), lambda i: (i, 0)),
        ],
        out_shape=[
            jax.ShapeDtypeStruct((NP, 64), jnp.float32),
            jax.ShapeDtypeStruct((NP, 64), jnp.bfloat16),
        ],
    )(x_pad, W1, deg1)

    b2, deg2, h1raw = pl.pallas_call(
        _mm_body,
        grid=(KB, KB),
        in_specs=[
            pl.BlockSpec((BK, NP), lambda i, j: (i, 0)),
            pl.BlockSpec((NP, BK), lambda i, j: (0, j)),
            pl.BlockSpec((NP, 64), lambda i, j: (0, 0)),
        ],
        out_specs=[
            pl.BlockSpec((BK, BK), lambda i, j: (i, j)),
            pl.BlockSpec((BK, 128), lambda i, j: (i, 0)),
            pl.BlockSpec((BK, 64), lambda i, j: (i, 0)),
        ],
        out_shape=[
            jax.ShapeDtypeStruct((NP, NP), jnp.int8),
            jax.ShapeDtypeStruct((NP, 128), jnp.float32),
            jax.ShapeDtypeStruct((NP, 64), jnp.float32),
        ],
        compiler_params=pltpu.CompilerParams(
            dimension_semantics=("parallel", "arbitrary")),
    )(abf, abf, hp1)

    hp2 = pl.pallas_call(
        _hp2_body,
        grid=(KB,),
        in_specs=[
            pl.BlockSpec((BK, 64), lambda i: (i, 0)),
            pl.BlockSpec((BK, 128), lambda i: (i, 0)),
        ],
        out_specs=pl.BlockSpec((BK, 64), lambda i: (i, 0)),
        out_shape=jax.ShapeDtypeStruct((NP, 64), jnp.bfloat16),
    )(h, deg2)

    h2raw = pl.pallas_call(
        _h2_body,
        grid=(KB,),
        in_specs=[
            pl.BlockSpec((BK, NP), lambda i: (i, 0)),
            pl.BlockSpec((NP, 64), lambda i: (0, 0)),
        ],
        out_specs=pl.BlockSpec((BK, 64), lambda i: (i, 0)),
        out_shape=jax.ShapeDtypeStruct((NP, 64), jnp.float32),
    )(b2, hp2)

    out = pl.pallas_call(
        _final_body,
        grid=(KB,),
        in_specs=[
            pl.BlockSpec((BK, 64), lambda i: (i, 0)),
            pl.BlockSpec((BK, 64), lambda i: (i, 0)),
            pl.BlockSpec((BK, 64), lambda i: (i, 0)),
            pl.BlockSpec((BK, 128), lambda i: (i, 0)),
            pl.BlockSpec((BK, 128), lambda i: (i, 0)),
            pl.BlockSpec((192, 64), lambda i: (0, 0)),
        ],
        out_specs=pl.BlockSpec((BK, 64), lambda i: (i, 0)),
        out_shape=jax.ShapeDtypeStruct((NP, 64), jnp.float32),
    )(h, h1raw, h2raw, deg1, deg2, W_final)

    return out[:N]
